# Initial kernel scaffold; baseline (speedup 1.0000x reference)
import jax
import jax.numpy as jnp
from jax.experimental import pallas as pl


def kernel(x, edge_index, W1, b1, W2, b2, W3, b3, W4, b4, W5, b5, W9, b9):
    raise NotImplementedError("write your pallas kernel here")


# speedup vs baseline: 10.1356x; 10.1356x over previous
import functools

import jax
import jax.numpy as jnp
from jax import lax
from jax.experimental import pallas as pl
from jax.experimental.pallas import tpu as pltpu
from jax.experimental.pallas import tpu_sc as plsc

N = 10000
NP = 10240
E = 320000
CHUNK = 128
NC = 2
NS = 16
NCH32 = 79
NCH16 = 158
EP = 32 * NCH32 * CHUNK
ROWS_PER_TILE = NP // NS


@functools.lru_cache(maxsize=None)
def _make_prop_split():
  mesh = plsc.VectorSubcoreMesh(core_axis_name="c", subcore_axis_name="s")

  @functools.partial(
      pl.kernel,
      out_type=jax.ShapeDtypeStruct((NC, NP, 64), jnp.float32),
      mesh=mesh,
      scratch_types=[
          pltpu.VMEM((NCH16, CHUNK), jnp.int32),
          pltpu.VMEM((NCH16, CHUNK), jnp.int32),
          pltpu.VMEM((CHUNK, 64), jnp.float32),
          pltpu.VMEM((ROWS_PER_TILE, 64), jnp.float32),
          pltpu.VMEM_SHARED((NP, 64), jnp.float32),
          pltpu.SemaphoreType.DMA,
      ],
      compiler_params=pltpu.CompilerParams(use_tc_tiling_on_sc=False),
  )
  def prop(src_hbm, dst_hbm, table_hbm, zeros_hbm, out_hbm,
           src_v, dst_v, rows_v, buf_v, acc_sh, sem):
    cid = lax.axis_index("c")
    sid = lax.axis_index("s")

    pltpu.sync_copy(src_hbm.at[sid], src_v)
    pltpu.sync_copy(dst_hbm.at[sid], dst_v)

    pltpu.sync_copy(zeros_hbm, buf_v)
    pltpu.sync_copy(buf_v, acc_sh.at[pl.ds(sid * ROWS_PER_TILE, ROWS_PER_TILE)])
    plsc.subcore_barrier()

    @pl.loop(0, NCH16)
    def _(j):
      pltpu.async_copy(table_hbm.at[cid].at[src_v.at[j]], rows_v, sem).wait()
      pltpu.sync_copy(rows_v, acc_sh.at[dst_v.at[j]], add=True)

    plsc.subcore_barrier()
    pltpu.sync_copy(acc_sh.at[pl.ds(sid * ROWS_PER_TILE, ROWS_PER_TILE)], buf_v)
    pltpu.sync_copy(buf_v, out_hbm.at[cid, pl.ds(sid * ROWS_PER_TILE,
                                                 ROWS_PER_TILE)])

  return prop


@functools.lru_cache(maxsize=None)
def _make_prop16():
  mesh = plsc.VectorSubcoreMesh(core_axis_name="c", subcore_axis_name="s")

  @functools.partial(
      pl.kernel,
      out_type=jax.ShapeDtypeStruct((NC, NP, 16), jnp.float32),
      mesh=mesh,
      scratch_types=[
          pltpu.VMEM((NCH32, CHUNK), jnp.int32),
          pltpu.VMEM((NCH32, CHUNK), jnp.int32),
          pltpu.VMEM((CHUNK, 16), jnp.float32),
          pltpu.VMEM((ROWS_PER_TILE, 16), jnp.float32),
          pltpu.VMEM_SHARED((NP, 16), jnp.float32),
          pltpu.SemaphoreType.DMA,
      ],
      compiler_params=pltpu.CompilerParams(use_tc_tiling_on_sc=False),
  )
  def prop(src_hbm, dst_hbm, table_hbm, zeros_hbm, out_hbm,
           src_v, dst_v, rows_v, buf_v, acc_sh, sem):
    cid = lax.axis_index("c")
    sid = lax.axis_index("s")
    wid = cid * NS + sid

    pltpu.sync_copy(src_hbm.at[wid], src_v)
    pltpu.sync_copy(dst_hbm.at[wid], dst_v)

    pltpu.sync_copy(zeros_hbm, buf_v)
    pltpu.sync_copy(buf_v, acc_sh.at[pl.ds(sid * ROWS_PER_TILE, ROWS_PER_TILE)])
    plsc.subcore_barrier()

    @pl.loop(0, NCH32)
    def _(j):
      pltpu.async_copy(table_hbm.at[src_v.at[j]], rows_v, sem).wait()
      pltpu.sync_copy(rows_v, acc_sh.at[dst_v.at[j]], add=True)

    plsc.subcore_barrier()
    pltpu.sync_copy(acc_sh.at[pl.ds(sid * ROWS_PER_TILE, ROWS_PER_TILE)], buf_v)
    pltpu.sync_copy(buf_v, out_hbm.at[cid, pl.ds(sid * ROWS_PER_TILE,
                                                 ROWS_PER_TILE)])

  return prop


def _tc_first(p16_ref, mask_ref, x_ref, w_ref, dinv_ref, h_ref):
  deg = p16_ref[0][:, 0:1] + p16_ref[1][:, 0:1] + 1.0
  dinv = lax.rsqrt(deg) * mask_ref[...]
  dinv_ref[...] = dinv
  z = dinv * jnp.dot(x_ref[...], w_ref[...],
                     preferred_element_type=jnp.float32)
  h_ref[0] = z[:, :64]
  h_ref[1] = z[:, 64:]


def _tc_mid(s_ref, h_ref, dinv_ref, b_ref, w_ref, o_ref):
  dinv = dinv_ref[...]
  full = jnp.concatenate([s_ref[0] + h_ref[0], s_ref[1] + h_ref[1]], axis=1)
  u = dinv * full + b_ref[...]
  a = jnp.maximum(u, 0.0)
  z = dinv * jnp.dot(a, w_ref[...], preferred_element_type=jnp.float32)
  o_ref[0] = z[:, :64]
  o_ref[1] = z[:, 64:]


def _tc_last(s_ref, h_ref, dinv_ref, b_ref, w_ref, o_ref):
  dinv = dinv_ref[...]
  full = jnp.concatenate([s_ref[0] + h_ref[0], s_ref[1] + h_ref[1]], axis=1)
  u = dinv * full + b_ref[...]
  a = jnp.maximum(u, 0.0)
  o_ref[...] = dinv * jnp.dot(a, w_ref[...],
                              preferred_element_type=jnp.float32)


def _tc_out(p_ref, h_ref, dinv_ref, b_ref, o_ref):
  o_ref[...] = (dinv_ref[...] * (p_ref[0] + p_ref[1] + h_ref[...])
                + b_ref[...])


def kernel(x, edge_index, W1, b1, W2, b2, W3, b3, W4, b4, W5, b5, W9, b9):
  f32 = jnp.float32

  src = edge_index[0].astype(jnp.int32)
  dst = edge_index[1].astype(jnp.int32)
  padlen = EP - E
  src = jnp.concatenate([src, jnp.full((padlen,), N, jnp.int32)])
  dst = jnp.concatenate([dst, jnp.full((padlen,), N, jnp.int32)])
  src32 = src.reshape(NC * NS, NCH32, CHUNK)
  dst32 = dst.reshape(NC * NS, NCH32, CHUNK)
  src16 = src.reshape(NS, NCH16, CHUNK)
  dst16 = dst.reshape(NS, NCH16, CHUNK)

  xp = jnp.zeros((NP, 128), f32).at[:N].set(x)
  mask = (jnp.arange(NP) < N).astype(f32)[:, None]
  ones16 = jnp.zeros((NP, 16), f32).at[:N].set(1.0)
  zeros64 = jnp.zeros((ROWS_PER_TILE, 64), f32)
  zeros16 = jnp.zeros((ROWS_PER_TILE, 16), f32)
  W9p = jnp.zeros((128, 16), f32).at[:, :5].set(W9)
  b9p = jnp.zeros((16,), f32).at[:5].set(b9)

  prop_split = _make_prop_split()
  prop16 = _make_prop16()

  p_deg = prop16(src32, dst32, ones16, zeros16)

  dinv, h = pl.pallas_call(
      _tc_first,
      out_shape=(jax.ShapeDtypeStruct((NP, 1), f32),
                 jax.ShapeDtypeStruct((NC, NP, 64), f32)),
  )(p_deg, mask, xp, W1)

  for W_next, b_prev in ((W2, b1), (W3, b2), (W4, b3), (W5, b4)):
    s = prop_split(src16, dst16, h, zeros64)
    h = pl.pallas_call(
        _tc_mid,
        out_shape=jax.ShapeDtypeStruct((NC, NP, 64), f32),
    )(s, h, dinv, b_prev.reshape(1, 128), W_next)

  s = prop_split(src16, dst16, h, zeros64)
  h6 = pl.pallas_call(
      _tc_last,
      out_shape=jax.ShapeDtypeStruct((NP, 16), f32),
  )(s, h, dinv, b5.reshape(1, 128), W9p)

  p6 = prop16(src32, dst32, h6, zeros16)
  out = pl.pallas_call(
      _tc_out,
      out_shape=jax.ShapeDtypeStruct((NP, 16), f32),
  )(p6, h6, dinv, b9p.reshape(1, 16))
  return out[:N, :5]
